# baseline (speedup 1.0000x reference)
import functools

import jax
import jax.numpy as jnp
from jax import lax
from jax.experimental import pallas as pl
from jax.experimental.pallas import tpu as pltpu
from jax.experimental.pallas import tpu_sc as plsc

D = 128
NC = 2
NS = 16
NW = NC * NS
CHUNK = 128
NBUF = 4


def kernel(action_ids, emb_weight):
    B0, S = action_ids.shape
    B = B0 * S
    flat = action_ids.reshape(B).astype(jnp.int32)

    per_w = B // NW
    n_chunks = per_w // CHUNK
    assert per_w * NW == B and n_chunks * CHUNK == per_w

    mesh = plsc.VectorSubcoreMesh(core_axis_name="c", subcore_axis_name="s")

    @functools.partial(
        pl.kernel,
        mesh=mesh,
        out_type=jax.ShapeDtypeStruct((B, D), jnp.float32),
        scratch_types=[
            pltpu.VMEM((NBUF, CHUNK), jnp.int32),
            pltpu.VMEM((NBUF, CHUNK, D), jnp.float32),
            pltpu.SemaphoreType.DMA((NBUF,)),
            pltpu.SemaphoreType.DMA,
            pltpu.SemaphoreType.DMA((NBUF,)),
        ],
    )
    def sc_embed(idx_hbm, table_hbm, out_hbm, idx_v, rows_v, sem_i, sem_g, sem_o):
        wid = lax.axis_index("s") * NC + lax.axis_index("c")
        base = wid * per_w

        def start_idx(g, b):
            pltpu.async_copy(
                idx_hbm.at[pl.ds(base + g * CHUNK, CHUNK)],
                idx_v.at[b],
                sem_i.at[b],
            )

        def wait_idx(g, b):
            pltpu.make_async_copy(
                idx_hbm.at[pl.ds(base + g * CHUNK, CHUNK)],
                idx_v.at[b],
                sem_i.at[b],
            ).wait()

        def start_out(g, b):
            pltpu.async_copy(
                rows_v.at[b],
                out_hbm.at[pl.ds(base + g * CHUNK, CHUNK)],
                sem_o.at[b],
            )

        def wait_out(g, b):
            pltpu.make_async_copy(
                rows_v.at[b],
                out_hbm.at[pl.ds(base + g * CHUNK, CHUNK)],
                sem_o.at[b],
            ).wait()

        for b in range(NBUF):
            start_idx(b, b)

        def group(i, carry):
            g0 = i * NBUF
            for b in range(NBUF):
                g = g0 + b
                wait_idx(g, b)

                @pl.when(i > 0)
                def _():
                    wait_out(g - NBUF, b)

                pltpu.async_copy(
                    table_hbm.at[idx_v.at[b]], rows_v.at[b], sem_g
                ).wait()
                start_out(g, b)

                @pl.when(g + NBUF < n_chunks)
                def _():
                    start_idx(g + NBUF, b)
            return carry

        lax.fori_loop(0, n_chunks // NBUF, group, None)

        for b in range(NBUF):
            wait_out(n_chunks - NBUF + b, b)

    out = sc_embed(flat, emb_weight)
    return out.reshape(B0, S, D)


# speedup vs baseline: 2.2902x; 2.2902x over previous
import functools

import jax
import jax.numpy as jnp
from jax import lax
from jax.experimental import pallas as pl
from jax.experimental.pallas import tpu as pltpu
from jax.experimental.pallas import tpu_sc as plsc

D = 128
V = 4
NC = 2
NS = 16
NW = NC * NS
CHUNK = 400
NBUF = 2
L = 16


def kernel(action_ids, emb_weight):
    B0, S = action_ids.shape
    B = B0 * S
    flat_idx = action_ids.reshape(B).astype(jnp.int32)

    per_w = B // NW
    n_chunks = per_w // CHUNK
    assert per_w * NW == B and n_chunks * CHUNK == per_w and n_chunks % NBUF == 0

    mesh = plsc.VectorSubcoreMesh(core_axis_name="c", subcore_axis_name="s")

    @functools.partial(
        pl.kernel,
        mesh=mesh,
        out_type=jax.ShapeDtypeStruct((B, D), jnp.float32),
        compiler_params=pltpu.CompilerParams(needs_layout_passes=False),
        scratch_types=[
            pltpu.VMEM((V, D), jnp.float32),
            pltpu.VMEM((NBUF * CHUNK,), jnp.int32),
            pltpu.VMEM((NBUF * CHUNK, D), jnp.float32),
            pltpu.SemaphoreType.DMA,
            pltpu.SemaphoreType.DMA((NBUF,)),
            pltpu.SemaphoreType.DMA((NBUF,)),
        ],
    )
    def sc_embed(idx_hbm, tab_hbm, out_hbm, tab_v, idx_v, rows_v,
                 sem_t, sem_i, sem_o):
        wid = lax.axis_index("s") * NC + lax.axis_index("c")
        base = wid * per_w

        iota = lax.iota(jnp.int32, L)

        def start_idx(g, b):
            pltpu.async_copy(
                idx_hbm.at[pl.ds(base + g * CHUNK, CHUNK)],
                idx_v.at[pl.ds(b * CHUNK, CHUNK)],
                sem_i.at[b],
            )

        def wait_idx(g, b):
            pltpu.make_async_copy(
                idx_hbm.at[pl.ds(base + g * CHUNK, CHUNK)],
                idx_v.at[pl.ds(b * CHUNK, CHUNK)],
                sem_i.at[b],
            ).wait()

        def start_out(g, b):
            pltpu.async_copy(
                rows_v.at[pl.ds(b * CHUNK, CHUNK)],
                out_hbm.at[pl.ds(base + g * CHUNK, CHUNK)],
                sem_o.at[b],
            )

        def wait_out(g, b):
            pltpu.make_async_copy(
                rows_v.at[pl.ds(b * CHUNK, CHUNK)],
                out_hbm.at[pl.ds(base + g * CHUNK, CHUNK)],
                sem_o.at[b],
            ).wait()

        def expand(b):
            def grp(k, carry):
                v16 = idx_v[pl.ds(b * CHUNK + k * L, L)]
                rows = iota + (b * CHUNK + k * L)
                for j in range(D):
                    col = jnp.full((L,), j, jnp.int32)
                    vals = plsc.load_gather(tab_v, [v16, col])
                    plsc.store_scatter(rows_v, [rows, col], vals)
                return carry
            lax.fori_loop(0, CHUNK // L, grp, None)

        pltpu.async_copy(tab_hbm, tab_v, sem_t).wait()
        for b in range(NBUF):
            start_idx(b, b)

        def group(i, carry):
            g0 = i * NBUF
            for b in range(NBUF):
                g = g0 + b
                wait_idx(g, b)

                @pl.when(i > 0)
                def _():
                    wait_out(g - NBUF, b)

                expand(b)
                start_out(g, b)

                @pl.when(g + NBUF < n_chunks)
                def _():
                    start_idx(g + NBUF, b)
            return carry

        lax.fori_loop(0, n_chunks // NBUF, group, None)

        for b in range(NBUF):
            wait_out(n_chunks - NBUF + b, b)

    out = sc_embed(flat_idx, emb_weight)
    return out.reshape(B0, S, D)


# speedup vs baseline: 3.6581x; 1.5973x over previous
import functools

import jax
import jax.numpy as jnp
from jax import lax
from jax.experimental import pallas as pl
from jax.experimental.pallas import tpu as pltpu
from jax.experimental.pallas import tpu_sc as plsc

D = 128
V = 4
NC = 2
NS = 16
NW = NC * NS
CHUNK = 400
NBUF = 2
L = 16


def kernel(action_ids, emb_weight):
    B0, S = action_ids.shape
    B = B0 * S
    flat_idx = action_ids.reshape(B).astype(jnp.int32)

    per_w = B // NW
    n_chunks = per_w // CHUNK
    assert per_w * NW == B and n_chunks * CHUNK == per_w and n_chunks % NBUF == 0

    mesh = plsc.VectorSubcoreMesh(core_axis_name="c", subcore_axis_name="s")

    @functools.partial(
        pl.kernel,
        mesh=mesh,
        out_type=jax.ShapeDtypeStruct((B, D), jnp.float32),
        compiler_params=pltpu.CompilerParams(needs_layout_passes=False),
        scratch_types=[
            pltpu.VMEM((V, D), jnp.float32),
            pltpu.VMEM((NBUF * CHUNK,), jnp.int32),
            pltpu.VMEM((NBUF * CHUNK, D), jnp.float32),
            pltpu.SemaphoreType.DMA,
            pltpu.SemaphoreType.DMA((NBUF,)),
            pltpu.SemaphoreType.DMA((NBUF,)),
        ],
    )
    def sc_embed(idx_hbm, tab_hbm, out_hbm, tab_v, idx_v, rows_v,
                 sem_t, sem_i, sem_o):
        wid = lax.axis_index("s") * NC + lax.axis_index("c")
        base = wid * per_w

        iota = lax.iota(jnp.int32, L)

        def start_idx(g, b):
            pltpu.async_copy(
                idx_hbm.at[pl.ds(base + g * CHUNK, CHUNK)],
                idx_v.at[pl.ds(b * CHUNK, CHUNK)],
                sem_i.at[b],
            )

        def wait_idx(g, b):
            pltpu.make_async_copy(
                idx_hbm.at[pl.ds(base + g * CHUNK, CHUNK)],
                idx_v.at[pl.ds(b * CHUNK, CHUNK)],
                sem_i.at[b],
            ).wait()

        def start_out(g, b):
            pltpu.async_copy(
                rows_v.at[pl.ds(b * CHUNK, CHUNK)],
                out_hbm.at[pl.ds(base + g * CHUNK, CHUNK)],
                sem_o.at[b],
            )

        def wait_out(g, b):
            pltpu.make_async_copy(
                rows_v.at[pl.ds(b * CHUNK, CHUNK)],
                out_hbm.at[pl.ds(base + g * CHUNK, CHUNK)],
                sem_o.at[b],
            ).wait()

        def expand(b):
            def grp(k, carry):
                v16 = idx_v[pl.ds(b * CHUNK + k * L, L)]
                rows = iota + (b * CHUNK + k * L)

                @plsc.parallel_loop(0, D, 1, unroll=8)
                def _(j):
                    col = jnp.broadcast_to(j, (L,)).astype(jnp.int32)
                    vals = plsc.load_gather(tab_v, [v16, col])
                    plsc.store_scatter(rows_v, [rows, col], vals)

                return carry
            lax.fori_loop(0, CHUNK // L, grp, None)

        pltpu.async_copy(tab_hbm, tab_v, sem_t).wait()
        for b in range(NBUF):
            start_idx(b, b)

        def group(i, carry):
            g0 = i * NBUF
            for b in range(NBUF):
                g = g0 + b
                wait_idx(g, b)

                @pl.when(i > 0)
                def _():
                    wait_out(g - NBUF, b)

                expand(b)
                start_out(g, b)

                @pl.when(g + NBUF < n_chunks)
                def _():
                    start_idx(g + NBUF, b)
            return carry

        lax.fori_loop(0, n_chunks // NBUF, group, None)

        for b in range(NBUF):
            wait_out(n_chunks - NBUF + b, b)

    out = sc_embed(flat_idx, emb_weight)
    return out.reshape(B0, S, D)


# speedup vs baseline: 50.8002x; 13.8869x over previous
import functools

import jax
import jax.numpy as jnp
from jax import lax
from jax.experimental import pallas as pl
from jax.experimental.pallas import tpu as pltpu
from jax.experimental.pallas import tpu_sc as plsc

D = 128
V = 4
NC = 2
NS = 16
NW = NC * NS
CHUNK = 400
NBUF = 2
L = 16


def kernel(action_ids, emb_weight):
    B0, S = action_ids.shape
    B = B0 * S
    flat_idx = action_ids.reshape(B).astype(jnp.int32)

    per_w = B // NW
    n_chunks = per_w // CHUNK
    assert per_w * NW == B and n_chunks * CHUNK == per_w and n_chunks % NBUF == 0

    mesh = plsc.VectorSubcoreMesh(core_axis_name="c", subcore_axis_name="s")

    @functools.partial(
        pl.kernel,
        mesh=mesh,
        out_type=jax.ShapeDtypeStruct((B, D), jnp.float32),
        compiler_params=pltpu.CompilerParams(needs_layout_passes=False),
        scratch_types=[
            pltpu.VMEM((V, D), jnp.float32),
            pltpu.VMEM((NBUF * CHUNK,), jnp.int32),
            pltpu.VMEM((NBUF * CHUNK, D), jnp.float32),
            pltpu.SemaphoreType.DMA,
            pltpu.SemaphoreType.DMA((NBUF,)),
            pltpu.SemaphoreType.DMA((NBUF,)),
        ],
    )
    def sc_embed(idx_hbm, tab_hbm, out_hbm, tab_v, idx_v, rows_v,
                 sem_t, sem_i, sem_o):
        wid = lax.axis_index("s") * NC + lax.axis_index("c")
        base = wid * per_w

        iota = lax.iota(jnp.int32, L)

        def start_idx(g, b):
            pltpu.async_copy(
                idx_hbm.at[pl.ds(base + g * CHUNK, CHUNK)],
                idx_v.at[pl.ds(b * CHUNK, CHUNK)],
                sem_i.at[b],
            )

        def wait_idx(g, b):
            pltpu.make_async_copy(
                idx_hbm.at[pl.ds(base + g * CHUNK, CHUNK)],
                idx_v.at[pl.ds(b * CHUNK, CHUNK)],
                sem_i.at[b],
            ).wait()

        def start_out(g, b):
            pltpu.async_copy(
                rows_v.at[pl.ds(b * CHUNK, CHUNK)],
                out_hbm.at[pl.ds(base + g * CHUNK, CHUNK)],
                sem_o.at[b],
            )

        def wait_out(g, b):
            pltpu.make_async_copy(
                rows_v.at[pl.ds(b * CHUNK, CHUNK)],
                out_hbm.at[pl.ds(base + g * CHUNK, CHUNK)],
                sem_o.at[b],
            ).wait()

        def expand(b):
            def grp(k, carry):
                row0 = b * CHUNK + k * L
                v16 = idx_v[pl.ds(row0, L)]

                @plsc.parallel_loop(0, L, 1, unroll=2)
                def _(i):
                    sel = jnp.broadcast_to(i, (L,)).astype(jnp.int32)
                    splat_vi = jnp.take_along_axis(v16, sel, axis=0)
                    for jj in range(D // L):
                        col = iota + (jj * L)
                        vals = plsc.load_gather(tab_v, [splat_vi, col])
                        rows_v[row0 + i, pl.ds(jj * L, L)] = vals

                return carry
            lax.fori_loop(0, CHUNK // L, grp, None)

        pltpu.async_copy(tab_hbm, tab_v, sem_t).wait()
        for b in range(NBUF):
            start_idx(b, b)

        def group(i, carry):
            g0 = i * NBUF
            for b in range(NBUF):
                g = g0 + b
                wait_idx(g, b)

                @pl.when(i > 0)
                def _():
                    wait_out(g - NBUF, b)

                expand(b)
                start_out(g, b)

                @pl.when(g + NBUF < n_chunks)
                def _():
                    start_idx(g + NBUF, b)
            return carry

        lax.fori_loop(0, n_chunks // NBUF, group, None)

        for b in range(NBUF):
            wait_out(n_chunks - NBUF + b, b)

    out = sc_embed(flat_idx, emb_weight)
    return out.reshape(B0, S, D)


# speedup vs baseline: 52.4914x; 1.0333x over previous
import functools

import jax
import jax.numpy as jnp
from jax import lax
from jax.experimental import pallas as pl
from jax.experimental.pallas import tpu as pltpu
from jax.experimental.pallas import tpu_sc as plsc

D = 128
V = 4
NC = 2
NS = 16
NW = NC * NS
CHUNK = 200
NBUF = 4
L = 16


def kernel(action_ids, emb_weight):
    B0, S = action_ids.shape
    B = B0 * S
    flat_idx = action_ids.reshape(B).astype(jnp.int32)

    per_w = B // NW
    n_chunks = per_w // CHUNK
    assert per_w * NW == B and n_chunks * CHUNK == per_w and n_chunks % NBUF == 0

    mesh = plsc.VectorSubcoreMesh(core_axis_name="c", subcore_axis_name="s")

    @functools.partial(
        pl.kernel,
        mesh=mesh,
        out_type=jax.ShapeDtypeStruct((B, D), jnp.float32),
        compiler_params=pltpu.CompilerParams(needs_layout_passes=False),
        scratch_types=[
            pltpu.VMEM((V, D), jnp.float32),
            pltpu.VMEM((NBUF * CHUNK,), jnp.int32),
            pltpu.VMEM((NBUF * CHUNK, D), jnp.float32),
            pltpu.SemaphoreType.DMA,
            pltpu.SemaphoreType.DMA((NBUF,)),
            pltpu.SemaphoreType.DMA((NBUF,)),
        ],
    )
    def sc_embed(idx_hbm, tab_hbm, out_hbm, tab_v, idx_v, rows_v,
                 sem_t, sem_i, sem_o):
        wid = lax.axis_index("s") * NC + lax.axis_index("c")
        base = wid * per_w

        iota = lax.iota(jnp.int32, L)

        def start_idx(g, b):
            pltpu.async_copy(
                idx_hbm.at[pl.ds(base + g * CHUNK, CHUNK)],
                idx_v.at[pl.ds(b * CHUNK, CHUNK)],
                sem_i.at[b],
            )

        def wait_idx(g, b):
            pltpu.make_async_copy(
                idx_hbm.at[pl.ds(base + g * CHUNK, CHUNK)],
                idx_v.at[pl.ds(b * CHUNK, CHUNK)],
                sem_i.at[b],
            ).wait()

        def start_out(g, b):
            pltpu.async_copy(
                rows_v.at[pl.ds(b * CHUNK, CHUNK)],
                out_hbm.at[pl.ds(base + g * CHUNK, CHUNK)],
                sem_o.at[b],
            )

        def wait_out(g, b):
            pltpu.make_async_copy(
                rows_v.at[pl.ds(b * CHUNK, CHUNK)],
                out_hbm.at[pl.ds(base + g * CHUNK, CHUNK)],
                sem_o.at[b],
            ).wait()

        def expand(b):
            @plsc.parallel_loop(0, CHUNK // L, 1, unroll=1)
            def grp(k):
                row0 = b * CHUNK + k * L
                v16 = idx_v[pl.ds(row0, L)]

                @plsc.parallel_loop(0, L, 1, unroll=2)
                def _(i):
                    sel = jnp.broadcast_to(i, (L,)).astype(jnp.int32)
                    splat_vi = jnp.take_along_axis(v16, sel, axis=0)
                    for jj in range(D // L):
                        col = iota + (jj * L)
                        vals = plsc.load_gather(tab_v, [splat_vi, col])
                        rows_v[row0 + i, pl.ds(jj * L, L)] = vals

        pltpu.async_copy(tab_hbm, tab_v, sem_t).wait()
        for b in range(NBUF):
            start_idx(b, b)

        def group(i, carry):
            g0 = i * NBUF
            for b in range(NBUF):
                g = g0 + b
                wait_idx(g, b)

                @pl.when(i > 0)
                def _():
                    wait_out(g - NBUF, b)

                expand(b)
                start_out(g, b)

                @pl.when(g + NBUF < n_chunks)
                def _():
                    start_idx(g + NBUF, b)
            return carry

        lax.fori_loop(0, n_chunks // NBUF, group, None)

        for b in range(NBUF):
            wait_out(n_chunks - NBUF + b, b)

    out = sc_embed(flat_idx, emb_weight)
    return out.reshape(B0, S, D)
